# baseline (speedup 1.0000x reference)
import functools

import jax
import jax.numpy as jnp
from jax.experimental import pallas as pl
from jax.experimental.pallas import tpu as pltpu
from jax.experimental.pallas import tpu_sc as plsc

E = 64
D = 768
H = 768
O = 768
B = 4096
CAP = 128
EPAD = 4
BT = 512
NB = B // BT
XROWS = (E + EPAD) * CAP
DP = D // 2
NC = 2
NS = 16
NW = NC * NS
TPW = B // NW


def _router_body(x_ref, gw_ref, gb_ref, eb_ref, slot_ref, xp_ref, counts_ref,
                 tri_ref):
    step = pl.program_id(0)

    @pl.when(step == 0)
    def _():
        counts_ref[...] = jnp.zeros_like(counts_ref)
        r = jax.lax.broadcasted_iota(jnp.int32, (BT, BT), 0)
        c = jax.lax.broadcasted_iota(jnp.int32, (BT, BT), 1)
        tri_ref[...] = (r >= c).astype(jnp.bfloat16)

    xf = x_ref[...]
    xb = xf.astype(jnp.bfloat16)
    lo = jax.lax.bitcast_convert_type(xb[:, :DP], jnp.uint16)
    hi = jax.lax.bitcast_convert_type(xb[:, DP:], jnp.uint16)
    xp_ref[...] = (lo.astype(jnp.uint32)
                   | (hi.astype(jnp.uint32) << 16)).astype(jnp.int32)
    scores = jnp.dot(xf, gw_ref[...], preferred_element_type=jnp.float32)
    scores = scores + gb_ref[...] + eb_ref[...]
    lane = jax.lax.broadcasted_iota(jnp.int32, (BT, E), 1).astype(jnp.float32)
    m = jnp.max(scores, axis=1, keepdims=True)
    eid = jnp.min(jnp.where(scores == m, lane, float(E)), axis=1,
                  keepdims=True)
    oh = lane == eid
    incl = jnp.dot(tri_ref[...], oh.astype(jnp.bfloat16),
                   preferred_element_type=jnp.float32)
    tot = incl + counts_ref[...]
    pos = jnp.sum(jnp.where(oh, tot, 0.0), axis=1, keepdims=True) - 1.0
    slot = jnp.where(pos < CAP, eid * CAP + pos, float(E * CAP))
    slot_ref[...] = slot.astype(jnp.int32)
    counts_ref[...] = tot[BT - 1:BT, :]


def _router(x, gate_W, gb2, eb2, interpret=False):
    return pl.pallas_call(
        _router_body,
        grid=(NB,),
        in_specs=[
            pl.BlockSpec((BT, D), lambda i: (i, 0)),
            pl.BlockSpec((D, E), lambda i: (0, 0)),
            pl.BlockSpec((1, E), lambda i: (0, 0)),
            pl.BlockSpec((1, E), lambda i: (0, 0)),
        ],
        out_specs=[
            pl.BlockSpec((BT, 1), lambda i: (i, 0)),
            pl.BlockSpec((BT, DP), lambda i: (i, 0)),
        ],
        out_shape=[
            jax.ShapeDtypeStruct((B, 1), jnp.int32),
            jax.ShapeDtypeStruct((B, DP), jnp.int32),
        ],
        scratch_shapes=[pltpu.VMEM((1, E), jnp.float32),
                        pltpu.VMEM((BT, BT), jnp.bfloat16)],
        interpret=interpret,
    )(x, gate_W, gb2, eb2)


EP = 4


def _mlp_body(xe_ref, w1_ref, b1_ref, w2_ref, b2_ref, ye_ref):
    g = pl.program_id(0)

    @pl.when(g < E // EP)
    def _():
        for u in range(EP):
            ec = jnp.minimum(g * EP + u, E - 1)
            xpk = xe_ref[pl.ds(u * CAP, CAP), :].astype(jnp.uint32)
            lo = jax.lax.bitcast_convert_type(
                (xpk & 0xFFFF).astype(jnp.uint16), jnp.bfloat16)
            hi = jax.lax.bitcast_convert_type(
                (xpk >> 16).astype(jnp.uint16), jnp.bfloat16)
            xb = jnp.concatenate([lo, hi], axis=1)
            w1 = w1_ref[u].astype(jnp.bfloat16)
            b1v = b1_ref[pl.ds(ec, 1), :]
            h = jnp.dot(xb, w1, preferred_element_type=jnp.float32)
            h = jnp.maximum(h + b1v, 0.0)
            w2 = w2_ref[u].astype(jnp.bfloat16)
            b2v = b2_ref[pl.ds(ec, 1), :]
            y = jnp.dot(h.astype(jnp.bfloat16), w2,
                        preferred_element_type=jnp.float32)
            ye_ref[pl.ds(u * CAP, CAP), :] = y + b2v

    @pl.when(g == E // EP)
    def _():
        ye_ref[...] = jnp.zeros_like(ye_ref)


def _mlp(xep, W1, b1, W2, b2, interpret=False):
    nw = E // EP - 1
    return pl.pallas_call(
        _mlp_body,
        grid=(E // EP + 1,),
        in_specs=[
            pl.BlockSpec((EP * CAP, DP), lambda g: (g, 0)),
            pl.BlockSpec((EP, D, H), lambda g: (jnp.minimum(g, nw), 0, 0)),
            pl.BlockSpec((E, H), lambda g: (0, 0)),
            pl.BlockSpec((EP, H, O), lambda g: (jnp.minimum(g, nw), 0, 0)),
            pl.BlockSpec((E, O), lambda g: (0, 0)),
        ],
        out_specs=pl.BlockSpec((EP * CAP, O), lambda g: (g, 0)),
        out_shape=jax.ShapeDtypeStruct((XROWS, O), jnp.float32),
        compiler_params=pltpu.CompilerParams(
            dimension_semantics=("arbitrary",)),
        interpret=interpret,
    )(xep, W1, b1, W2, b2)


def _dispatch(xp, slots2):
    mesh = plsc.VectorSubcoreMesh(core_axis_name="c", subcore_axis_name="s")

    @functools.partial(
        pl.kernel,
        out_type=jax.ShapeDtypeStruct((XROWS, DP), jnp.int32),
        mesh=mesh,
        scratch_types=[
            pltpu.VMEM((1, TPW), jnp.int32),
            pltpu.VMEM((TPW, DP), jnp.int32),
            pltpu.SemaphoreType.DMA,
        ],
    )
    def k(x_hbm, slot_hbm, xe_hbm, idx_v, rows_v, sem):
        wid = jax.lax.axis_index("s") * NC + jax.lax.axis_index("c")
        pltpu.sync_copy(slot_hbm.at[pl.ds(wid, 1)], idx_v)
        pltpu.sync_copy(x_hbm.at[pl.ds(wid * TPW, TPW)], rows_v)
        pltpu.async_copy(rows_v, xe_hbm.at[idx_v.at[0]], sem).wait()

    return k(xp, slots2)


def _combine(ye, slots2):
    mesh = plsc.VectorSubcoreMesh(core_axis_name="c", subcore_axis_name="s")

    @functools.partial(
        pl.kernel,
        out_type=jax.ShapeDtypeStruct((B, O), jnp.float32),
        mesh=mesh,
        scratch_types=[
            pltpu.VMEM((1, TPW), jnp.int32),
            pltpu.VMEM((TPW, O), jnp.float32),
            pltpu.SemaphoreType.DMA,
        ],
    )
    def k(ye_hbm, slot_hbm, out_hbm, idx_v, rows_v, sem):
        wid = jax.lax.axis_index("s") * NC + jax.lax.axis_index("c")
        pltpu.sync_copy(slot_hbm.at[pl.ds(wid, 1)], idx_v)
        pltpu.async_copy(ye_hbm.at[idx_v.at[0]], rows_v, sem).wait()
        pltpu.sync_copy(rows_v, out_hbm.at[pl.ds(wid * TPW, TPW)])

    return k(ye, slots2)


def kernel(x, gate_W, gate_b, W1, b1, W2, b2, expert_bias):
    slots, xp = _router(x, gate_W, gate_b.reshape(1, E),
                        expert_bias.reshape(1, E))
    slots = slots.reshape(NW, TPW)
    xe = _dispatch(xp, slots)
    ye = _mlp(xe, W1, b1, W2, b2)
    return _combine(ye, slots)


# speedup vs baseline: 1.0056x; 1.0056x over previous
import functools

import jax
import jax.numpy as jnp
from jax.experimental import pallas as pl
from jax.experimental.pallas import tpu as pltpu
from jax.experimental.pallas import tpu_sc as plsc

E = 64
D = 768
H = 768
O = 768
B = 4096
CAP = 128
EPAD = 2
BT = 512
NB = B // BT
XROWS = (E + EPAD) * CAP
DP = D // 2
NC = 2
NS = 16
NW = NC * NS
TPW = B // NW


def _router_body(x_ref, gw_ref, gb_ref, eb_ref, slot_ref, xp_ref, counts_ref,
                 tri_ref):
    step = pl.program_id(0)

    @pl.when(step == 0)
    def _():
        counts_ref[...] = jnp.zeros_like(counts_ref)
        r = jax.lax.broadcasted_iota(jnp.int32, (BT, BT), 0)
        c = jax.lax.broadcasted_iota(jnp.int32, (BT, BT), 1)
        tri_ref[...] = (r >= c).astype(jnp.bfloat16)

    xf = x_ref[...]
    xb = xf.astype(jnp.bfloat16)
    lo = jax.lax.bitcast_convert_type(xb[:, :DP], jnp.uint16)
    hi = jax.lax.bitcast_convert_type(xb[:, DP:], jnp.uint16)
    xp_ref[...] = (lo.astype(jnp.uint32)
                   | (hi.astype(jnp.uint32) << 16)).astype(jnp.int32)
    scores = jnp.dot(xf, gw_ref[...], preferred_element_type=jnp.float32)
    scores = scores + gb_ref[...] + eb_ref[...]
    lane = jax.lax.broadcasted_iota(jnp.int32, (BT, E), 1).astype(jnp.float32)
    m = jnp.max(scores, axis=1, keepdims=True)
    eid = jnp.min(jnp.where(scores == m, lane, float(E)), axis=1,
                  keepdims=True)
    oh = lane == eid
    incl = jnp.dot(tri_ref[...], oh.astype(jnp.bfloat16),
                   preferred_element_type=jnp.float32)
    tot = incl + counts_ref[...]
    pos = jnp.sum(jnp.where(oh, tot, 0.0), axis=1, keepdims=True) - 1.0
    slot = jnp.where(pos < CAP, eid * CAP + pos, float(E * CAP))
    slot_ref[...] = slot.astype(jnp.int32)
    counts_ref[...] = tot[BT - 1:BT, :]


def _router(x, gate_W, gb2, eb2, interpret=False):
    return pl.pallas_call(
        _router_body,
        grid=(NB,),
        in_specs=[
            pl.BlockSpec((BT, D), lambda i: (i, 0)),
            pl.BlockSpec((D, E), lambda i: (0, 0)),
            pl.BlockSpec((1, E), lambda i: (0, 0)),
            pl.BlockSpec((1, E), lambda i: (0, 0)),
        ],
        out_specs=[
            pl.BlockSpec((BT, 1), lambda i: (i, 0)),
            pl.BlockSpec((BT, DP), lambda i: (i, 0)),
        ],
        out_shape=[
            jax.ShapeDtypeStruct((B, 1), jnp.int32),
            jax.ShapeDtypeStruct((B, DP), jnp.int32),
        ],
        scratch_shapes=[pltpu.VMEM((1, E), jnp.float32),
                        pltpu.VMEM((BT, BT), jnp.bfloat16)],
        interpret=interpret,
    )(x, gate_W, gb2, eb2)


EP = 2


def _mlp_body(xe_ref, w1_ref, b1_ref, w2_ref, b2_ref, ye_ref):
    g = pl.program_id(0)

    @pl.when(g < E // EP)
    def _():
        for u in range(EP):
            ec = jnp.minimum(g * EP + u, E - 1)
            xpk = xe_ref[pl.ds(u * CAP, CAP), :].astype(jnp.uint32)
            lo = jax.lax.bitcast_convert_type(
                (xpk & 0xFFFF).astype(jnp.uint16), jnp.bfloat16)
            hi = jax.lax.bitcast_convert_type(
                (xpk >> 16).astype(jnp.uint16), jnp.bfloat16)
            xb = jnp.concatenate([lo, hi], axis=1)
            w1 = w1_ref[u].astype(jnp.bfloat16)
            b1v = b1_ref[pl.ds(ec, 1), :]
            h = jnp.dot(xb, w1, preferred_element_type=jnp.float32)
            h = jnp.maximum(h + b1v, 0.0)
            w2 = w2_ref[u].astype(jnp.bfloat16)
            b2v = b2_ref[pl.ds(ec, 1), :]
            y = jnp.dot(h.astype(jnp.bfloat16), w2,
                        preferred_element_type=jnp.float32)
            ye_ref[pl.ds(u * CAP, CAP), :] = y + b2v

    @pl.when(g == E // EP)
    def _():
        ye_ref[...] = jnp.zeros_like(ye_ref)


def _mlp(xep, W1, b1, W2, b2, interpret=False):
    nw = E // EP - 1
    return pl.pallas_call(
        _mlp_body,
        grid=(E // EP + 1,),
        in_specs=[
            pl.BlockSpec((EP * CAP, DP), lambda g: (g, 0)),
            pl.BlockSpec((EP, D, H), lambda g: (jnp.minimum(g, nw), 0, 0)),
            pl.BlockSpec((E, H), lambda g: (0, 0)),
            pl.BlockSpec((EP, H, O), lambda g: (jnp.minimum(g, nw), 0, 0)),
            pl.BlockSpec((E, O), lambda g: (0, 0)),
        ],
        out_specs=pl.BlockSpec((EP * CAP, O), lambda g: (g, 0)),
        out_shape=jax.ShapeDtypeStruct((XROWS, O), jnp.float32),
        compiler_params=pltpu.CompilerParams(
            dimension_semantics=("arbitrary",)),
        interpret=interpret,
    )(xep, W1, b1, W2, b2)


def _dispatch(xp, slots2):
    mesh = plsc.VectorSubcoreMesh(core_axis_name="c", subcore_axis_name="s")

    @functools.partial(
        pl.kernel,
        out_type=jax.ShapeDtypeStruct((XROWS, DP), jnp.int32),
        mesh=mesh,
        scratch_types=[
            pltpu.VMEM((1, TPW), jnp.int32),
            pltpu.VMEM((TPW, DP), jnp.int32),
            pltpu.SemaphoreType.DMA,
        ],
    )
    def k(x_hbm, slot_hbm, xe_hbm, idx_v, rows_v, sem):
        wid = jax.lax.axis_index("s") * NC + jax.lax.axis_index("c")
        pltpu.sync_copy(slot_hbm.at[pl.ds(wid, 1)], idx_v)
        pltpu.sync_copy(x_hbm.at[pl.ds(wid * TPW, TPW)], rows_v)
        pltpu.async_copy(rows_v, xe_hbm.at[idx_v.at[0]], sem).wait()

    return k(xp, slots2)


def _combine(ye, slots2):
    mesh = plsc.VectorSubcoreMesh(core_axis_name="c", subcore_axis_name="s")

    @functools.partial(
        pl.kernel,
        out_type=jax.ShapeDtypeStruct((B, O), jnp.float32),
        mesh=mesh,
        scratch_types=[
            pltpu.VMEM((1, TPW), jnp.int32),
            pltpu.VMEM((TPW, O), jnp.float32),
            pltpu.SemaphoreType.DMA,
        ],
    )
    def k(ye_hbm, slot_hbm, out_hbm, idx_v, rows_v, sem):
        wid = jax.lax.axis_index("s") * NC + jax.lax.axis_index("c")
        pltpu.sync_copy(slot_hbm.at[pl.ds(wid, 1)], idx_v)
        pltpu.async_copy(ye_hbm.at[idx_v.at[0]], rows_v, sem).wait()
        pltpu.sync_copy(rows_v, out_hbm.at[pl.ds(wid * TPW, TPW)])

    return k(ye, slots2)


def kernel(x, gate_W, gate_b, W1, b1, W2, b2, expert_bias):
    slots, xp = _router(x, gate_W, gate_b.reshape(1, E),
                        expert_bias.reshape(1, E))
    slots = slots.reshape(NW, TPW)
    xe = _dispatch(xp, slots)
    ye = _mlp(xe, W1, b1, W2, b2)
    return _combine(ye, slots)


# speedup vs baseline: 1.0197x; 1.0140x over previous
import functools

import jax
import jax.numpy as jnp
from jax.experimental import pallas as pl
from jax.experimental.pallas import tpu as pltpu
from jax.experimental.pallas import tpu_sc as plsc

E = 64
D = 768
H = 768
O = 768
B = 4096
CAP = 128
EPAD = 2
BT = 1024
NB = B // BT
XROWS = (E + EPAD) * CAP
DP = D // 2
NC = 2
NS = 16
NW = NC * NS
TPW = B // NW


def _router_body(x_ref, gw_ref, gb_ref, eb_ref, slot_ref, xp_ref, counts_ref,
                 tri_ref):
    step = pl.program_id(0)

    @pl.when(step == 0)
    def _():
        counts_ref[...] = jnp.zeros_like(counts_ref)
        r = jax.lax.broadcasted_iota(jnp.int32, (BT, BT), 0)
        c = jax.lax.broadcasted_iota(jnp.int32, (BT, BT), 1)
        tri_ref[...] = (r >= c).astype(jnp.bfloat16)

    xf = x_ref[...]
    xb = xf.astype(jnp.bfloat16)
    lo = jax.lax.bitcast_convert_type(xb[:, :DP], jnp.uint16)
    hi = jax.lax.bitcast_convert_type(xb[:, DP:], jnp.uint16)
    xp_ref[...] = (lo.astype(jnp.uint32)
                   | (hi.astype(jnp.uint32) << 16)).astype(jnp.int32)
    scores = jnp.dot(xf, gw_ref[...], preferred_element_type=jnp.float32)
    scores = scores + gb_ref[...] + eb_ref[...]
    lane = jax.lax.broadcasted_iota(jnp.int32, (BT, E), 1).astype(jnp.float32)
    m = jnp.max(scores, axis=1, keepdims=True)
    eid = jnp.min(jnp.where(scores == m, lane, float(E)), axis=1,
                  keepdims=True)
    oh = lane == eid
    incl = jnp.dot(tri_ref[...], oh.astype(jnp.bfloat16),
                   preferred_element_type=jnp.float32)
    tot = incl + counts_ref[...]
    pos = jnp.sum(jnp.where(oh, tot, 0.0), axis=1, keepdims=True) - 1.0
    slot = jnp.where(pos < CAP, eid * CAP + pos, float(E * CAP))
    slot_ref[...] = slot.astype(jnp.int32)
    counts_ref[...] = tot[BT - 1:BT, :]


def _router(x, gate_W, gb2, eb2, interpret=False):
    return pl.pallas_call(
        _router_body,
        grid=(NB,),
        in_specs=[
            pl.BlockSpec((BT, D), lambda i: (i, 0)),
            pl.BlockSpec((D, E), lambda i: (0, 0)),
            pl.BlockSpec((1, E), lambda i: (0, 0)),
            pl.BlockSpec((1, E), lambda i: (0, 0)),
        ],
        out_specs=[
            pl.BlockSpec((BT, 1), lambda i: (i, 0)),
            pl.BlockSpec((BT, DP), lambda i: (i, 0)),
        ],
        out_shape=[
            jax.ShapeDtypeStruct((B, 1), jnp.int32),
            jax.ShapeDtypeStruct((B, DP), jnp.int32),
        ],
        scratch_shapes=[pltpu.VMEM((1, E), jnp.float32),
                        pltpu.VMEM((BT, BT), jnp.bfloat16)],
        interpret=interpret,
    )(x, gate_W, gb2, eb2)


EP = 2


def _mlp_body(xe_ref, w1_ref, b1_ref, w2_ref, b2_ref, ye_ref):
    g = pl.program_id(0)

    @pl.when(g < E // EP)
    def _():
        for u in range(EP):
            ec = jnp.minimum(g * EP + u, E - 1)
            xpk = xe_ref[pl.ds(u * CAP, CAP), :].astype(jnp.uint32)
            lo = jax.lax.bitcast_convert_type(
                (xpk & 0xFFFF).astype(jnp.uint16), jnp.bfloat16)
            hi = jax.lax.bitcast_convert_type(
                (xpk >> 16).astype(jnp.uint16), jnp.bfloat16)
            xb = jnp.concatenate([lo, hi], axis=1)
            w1 = w1_ref[u].astype(jnp.bfloat16)
            b1v = b1_ref[pl.ds(ec, 1), :]
            h = jnp.dot(xb, w1, preferred_element_type=jnp.float32)
            h = jnp.maximum(h + b1v, 0.0)
            w2 = w2_ref[u].astype(jnp.bfloat16)
            b2v = b2_ref[pl.ds(ec, 1), :]
            y = jnp.dot(h.astype(jnp.bfloat16), w2,
                        preferred_element_type=jnp.float32)
            ye_ref[pl.ds(u * CAP, CAP), :] = y + b2v

    @pl.when(g == E // EP)
    def _():
        ye_ref[...] = jnp.zeros_like(ye_ref)


def _mlp(xep, W1, b1, W2, b2, interpret=False):
    nw = E // EP - 1
    return pl.pallas_call(
        _mlp_body,
        grid=(E // EP + 1,),
        in_specs=[
            pl.BlockSpec((EP * CAP, DP), lambda g: (g, 0)),
            pl.BlockSpec((EP, D, H), lambda g: (jnp.minimum(g, nw), 0, 0)),
            pl.BlockSpec((E, H), lambda g: (0, 0)),
            pl.BlockSpec((EP, H, O), lambda g: (jnp.minimum(g, nw), 0, 0)),
            pl.BlockSpec((E, O), lambda g: (0, 0)),
        ],
        out_specs=pl.BlockSpec((EP * CAP, O), lambda g: (g, 0)),
        out_shape=jax.ShapeDtypeStruct((XROWS, O), jnp.float32),
        compiler_params=pltpu.CompilerParams(
            dimension_semantics=("arbitrary",)),
        interpret=interpret,
    )(xep, W1, b1, W2, b2)


def _dispatch(xp, slots2):
    mesh = plsc.VectorSubcoreMesh(core_axis_name="c", subcore_axis_name="s")

    @functools.partial(
        pl.kernel,
        out_type=jax.ShapeDtypeStruct((XROWS, DP), jnp.int32),
        mesh=mesh,
        scratch_types=[
            pltpu.VMEM((1, TPW), jnp.int32),
            pltpu.VMEM((TPW, DP), jnp.int32),
            pltpu.SemaphoreType.DMA,
        ],
    )
    def k(x_hbm, slot_hbm, xe_hbm, idx_v, rows_v, sem):
        wid = jax.lax.axis_index("s") * NC + jax.lax.axis_index("c")
        pltpu.sync_copy(slot_hbm.at[pl.ds(wid, 1)], idx_v)
        pltpu.sync_copy(x_hbm.at[pl.ds(wid * TPW, TPW)], rows_v)
        pltpu.async_copy(rows_v, xe_hbm.at[idx_v.at[0]], sem).wait()

    return k(xp, slots2)


HPW = TPW // 2


def _combine(ye, slots2):
    mesh = plsc.VectorSubcoreMesh(core_axis_name="c", subcore_axis_name="s")

    @functools.partial(
        pl.kernel,
        out_type=jax.ShapeDtypeStruct((B, O), jnp.float32),
        mesh=mesh,
        scratch_types=[
            pltpu.VMEM((1, TPW), jnp.int32),
            pltpu.VMEM((HPW, O), jnp.float32),
            pltpu.VMEM((HPW, O), jnp.float32),
            pltpu.SemaphoreType.DMA,
            pltpu.SemaphoreType.DMA,
            pltpu.SemaphoreType.DMA,
            pltpu.SemaphoreType.DMA,
        ],
    )
    def k(ye_hbm, slot_hbm, out_hbm, idx_v, rows0, rows1, s0, s1, s2, s3):
        wid = jax.lax.axis_index("s") * NC + jax.lax.axis_index("c")
        base = wid * TPW
        pltpu.sync_copy(slot_hbm.at[pl.ds(wid, 1)], idx_v)
        g0 = pltpu.async_copy(ye_hbm.at[idx_v.at[0, pl.ds(0, HPW)]], rows0, s0)
        g1 = pltpu.async_copy(ye_hbm.at[idx_v.at[0, pl.ds(HPW, HPW)]], rows1,
                              s1)
        g0.wait()
        o0 = pltpu.async_copy(rows0, out_hbm.at[pl.ds(base, HPW)], s2)
        g1.wait()
        o1 = pltpu.async_copy(rows1, out_hbm.at[pl.ds(base + HPW, HPW)], s3)
        o0.wait()
        o1.wait()

    return k(ye, slots2)


def kernel(x, gate_W, gate_b, W1, b1, W2, b2, expert_bias):
    slots, xp = _router(x, gate_W, gate_b.reshape(1, E),
                        expert_bias.reshape(1, E))
    slots = slots.reshape(NW, TPW)
    xe = _dispatch(xp, slots)
    ye = _mlp(xe, W1, b1, W2, b2)
    return _combine(ye, slots)
